# Initial kernel scaffold; baseline (speedup 1.0000x reference)
import functools

import jax
import jax.numpy as jnp
from jax import lax
from jax.experimental import pallas as pl
from jax.experimental.pallas import tpu as pltpu
from jax.experimental.pallas import tpu_sc as plsc

N = 10000
D = 128
E = 320000
NC = 2
NS = 16
NW = NC * NS
EPW = E // NW
C = 80
NCHUNK = EPW // C
NPT = N // NS

_mesh = plsc.VectorSubcoreMesh(core_axis_name="c", subcore_axis_name="s")


@functools.partial(
    pl.kernel,
    out_type=jax.ShapeDtypeStruct((NC, N, 16), jnp.float32),
    mesh=_mesh,
    scratch_types=[
        pltpu.VMEM((NCHUNK, C), jnp.int32),
        pltpu.VMEM((C, 16), jnp.float32),
        pltpu.VMEM((NPT, 16), jnp.float32),
        pltpu.VMEM_SHARED((N, 16), jnp.float32),
    ],
)
def _deg_kernel(dst_hbm, outp, idxv, onesv, zerov, deg_sh):
    cid = lax.axis_index("c")
    sid = lax.axis_index("s")

    ones16 = jnp.ones((16,), jnp.float32)
    zeros16 = jnp.zeros((16,), jnp.float32)

    def fill_ones(i, carry):
        onesv[i] = ones16
        return carry

    lax.fori_loop(0, C, fill_ones, 0)

    def fill_zeros(i, carry):
        zerov[i] = zeros16
        return carry

    lax.fori_loop(0, NPT, fill_zeros, 0)

    pltpu.sync_copy(zerov, deg_sh.at[pl.ds(sid * NPT, NPT)])
    plsc.subcore_barrier()

    pltpu.sync_copy(dst_hbm.at[cid, sid], idxv)

    def chunk(j, carry):
        pltpu.sync_copy(onesv, deg_sh.at[idxv.at[j]], add=True)
        return carry

    lax.fori_loop(0, NCHUNK, chunk, 0)
    plsc.subcore_barrier()

    pltpu.sync_copy(deg_sh.at[pl.ds(sid * NPT, NPT)],
                    outp.at[cid, pl.ds(sid * NPT, NPT)])


def _scale_body(x_ref, w_ref, degp_ref, g_ref):
    s = degp_ref[0] + degp_ref[1]
    deg = jnp.sum(s, axis=1) * (1.0 / 16.0) + 1.0
    dinv = lax.rsqrt(deg)
    h = jnp.dot(x_ref[...], w_ref[...], preferred_element_type=jnp.float32)
    g_ref[...] = h * dinv[:, None]


@functools.partial(
    pl.kernel,
    out_type=jax.ShapeDtypeStruct((NC, N, D), jnp.float32),
    mesh=_mesh,
    scratch_types=[
        pltpu.VMEM((NCHUNK, C), jnp.int32),
        pltpu.VMEM((NCHUNK, C), jnp.int32),
        pltpu.VMEM((C, D), jnp.float32),
        pltpu.VMEM((NCHUNK, D), jnp.float32),
        pltpu.VMEM_SHARED((N, D), jnp.float32),
        pltpu.SemaphoreType.DMA,
    ],
)
def _agg_kernel(src_hbm, dst_hbm, g_hbm, outp, srcv, dstv, rows, zerov,
                agg_sh, sem):
    cid = lax.axis_index("c")
    sid = lax.axis_index("s")

    zeros16 = jnp.zeros((16,), jnp.float32)

    def fill_zeros(i, carry):
        for c8 in range(D // 16):
            zerov[i, pl.ds(c8 * 16, 16)] = zeros16
        return carry

    lax.fori_loop(0, NCHUNK, fill_zeros, 0)
    for r in range(NPT // NCHUNK):
        pltpu.sync_copy(zerov, agg_sh.at[pl.ds(sid * NPT + r * NCHUNK, NCHUNK)])
    plsc.subcore_barrier()

    pltpu.sync_copy(src_hbm.at[cid, sid], srcv)
    pltpu.sync_copy(dst_hbm.at[cid, sid], dstv)

    def chunk(j, carry):
        pltpu.async_copy(g_hbm.at[srcv.at[j]], rows, sem).wait()
        pltpu.sync_copy(rows, agg_sh.at[dstv.at[j]], add=True)
        return carry

    lax.fori_loop(0, NCHUNK, chunk, 0)
    plsc.subcore_barrier()

    pltpu.sync_copy(agg_sh.at[pl.ds(sid * NPT, NPT)],
                    outp.at[cid, pl.ds(sid * NPT, NPT)])


def _combine_body(degp_ref, p_ref, g_ref, b_ref, out_ref):
    s = degp_ref[0] + degp_ref[1]
    deg = jnp.sum(s, axis=1) * (1.0 / 16.0) + 1.0
    dinv = lax.rsqrt(deg)
    acc = (p_ref[0] + p_ref[1] + g_ref[...]) * dinv[:, None] + b_ref[0][None, :]
    out_ref[...] = jnp.broadcast_to(acc[None], out_ref.shape)


BN = 400


def kernel(x, edge_index, W, b):
    ei = edge_index.astype(jnp.int32)
    src = ei[0].reshape(NC, NS, NCHUNK, C)
    dst = ei[1].reshape(NC, NS, NCHUNK, C)

    degp = _deg_kernel(dst)

    g = pl.pallas_call(
        _scale_body,
        grid=(N // BN,),
        in_specs=[
            pl.BlockSpec((BN, D), lambda i: (i, 0)),
            pl.BlockSpec((D, D), lambda i: (0, 0)),
            pl.BlockSpec((NC, BN, 16), lambda i: (0, i, 0)),
        ],
        out_specs=pl.BlockSpec((BN, D), lambda i: (i, 0)),
        out_shape=jax.ShapeDtypeStruct((N, D), jnp.float32),
    )(x, W, degp)

    partials = _agg_kernel(src, dst, g)

    out = pl.pallas_call(
        _combine_body,
        grid=(N // BN,),
        in_specs=[
            pl.BlockSpec((NC, BN, 16), lambda i: (0, i, 0)),
            pl.BlockSpec((NC, BN, D), lambda i: (0, i, 0)),
            pl.BlockSpec((BN, D), lambda i: (i, 0)),
            pl.BlockSpec((1, D), lambda i: (0, 0)),
        ],
        out_specs=pl.BlockSpec((5, BN, D), lambda i: (0, i, 0)),
        out_shape=jax.ShapeDtypeStruct((5, N, D), jnp.float32),
    )(degp, partials, g, b.reshape(1, D))

    return out


# speedup vs baseline: 25.0252x; 25.0252x over previous
import functools

import jax
import jax.numpy as jnp
from jax import lax
from jax.experimental import pallas as pl
from jax.experimental.pallas import tpu as pltpu
from jax.experimental.pallas import tpu_sc as plsc

N = 10000
D = 128
E = 320000
NC = 2
NS = 16
NW = NC * NS
EPW = E // NW
C = 80
NCHUNK = EPW // C
NPT = N // NS

_mesh = plsc.VectorSubcoreMesh(core_axis_name="c", subcore_axis_name="s")
_sc_params = pltpu.CompilerParams(use_tc_tiling_on_sc=False)


@functools.partial(
    pl.kernel,
    out_type=jax.ShapeDtypeStruct((NC, N, 16), jnp.float32),
    mesh=_mesh,
    scratch_types=[
        pltpu.VMEM((NCHUNK, C), jnp.int32),
        pltpu.VMEM((C, 16), jnp.float32),
        pltpu.VMEM((NPT, 16), jnp.float32),
        pltpu.VMEM_SHARED((N, 16), jnp.float32),
    ],
    compiler_params=_sc_params,
)
def _deg_kernel(dst_hbm, outp, idxv, onesv, zerov, deg_sh):
    cid = lax.axis_index("c")
    sid = lax.axis_index("s")

    ones16 = jnp.ones((16,), jnp.float32)
    zeros16 = jnp.zeros((16,), jnp.float32)

    def fill_ones(i, carry):
        onesv[i] = ones16
        return carry

    lax.fori_loop(0, C, fill_ones, 0)

    def fill_zeros(i, carry):
        zerov[i] = zeros16
        return carry

    lax.fori_loop(0, NPT, fill_zeros, 0)

    pltpu.sync_copy(zerov, deg_sh.at[pl.ds(sid * NPT, NPT)])
    plsc.subcore_barrier()

    pltpu.sync_copy(dst_hbm.at[cid, sid], idxv)

    def chunk(j, carry):
        pltpu.sync_copy(onesv, deg_sh.at[idxv.at[j]], add=True)
        return carry

    lax.fori_loop(0, NCHUNK, chunk, 0)
    plsc.subcore_barrier()

    pltpu.sync_copy(deg_sh.at[pl.ds(sid * NPT, NPT)],
                    outp.at[cid, pl.ds(sid * NPT, NPT)])


def _scale_body(x_ref, w_ref, degp_ref, g_ref):
    s = degp_ref[0] + degp_ref[1]
    deg = jnp.sum(s, axis=1) * (1.0 / 16.0) + 1.0
    dinv = lax.rsqrt(deg)
    h = jnp.dot(x_ref[...], w_ref[...], preferred_element_type=jnp.float32)
    g_ref[...] = h * dinv[:, None]


@functools.partial(
    pl.kernel,
    out_type=jax.ShapeDtypeStruct((NC, N, D), jnp.float32),
    mesh=_mesh,
    scratch_types=[
        pltpu.VMEM((NCHUNK, C), jnp.int32),
        pltpu.VMEM((NCHUNK, C), jnp.int32),
        pltpu.VMEM((C, D), jnp.float32),
        pltpu.VMEM((NCHUNK, D), jnp.float32),
        pltpu.VMEM_SHARED((N, D), jnp.float32),
        pltpu.SemaphoreType.DMA,
    ],
    compiler_params=_sc_params,
)
def _agg_kernel(src_hbm, dst_hbm, g_hbm, outp, srcv, dstv, rows, zerov,
                agg_sh, sem):
    cid = lax.axis_index("c")
    sid = lax.axis_index("s")

    zeros16 = jnp.zeros((16,), jnp.float32)

    def fill_zeros(i, carry):
        for c8 in range(D // 16):
            zerov[i, pl.ds(c8 * 16, 16)] = zeros16
        return carry

    lax.fori_loop(0, NCHUNK, fill_zeros, 0)
    for r in range(NPT // NCHUNK):
        pltpu.sync_copy(zerov, agg_sh.at[pl.ds(sid * NPT + r * NCHUNK, NCHUNK)])
    plsc.subcore_barrier()

    pltpu.sync_copy(src_hbm.at[cid, sid], srcv)
    pltpu.sync_copy(dst_hbm.at[cid, sid], dstv)

    def chunk(j, carry):
        pltpu.async_copy(g_hbm.at[srcv.at[j]], rows, sem).wait()
        pltpu.sync_copy(rows, agg_sh.at[dstv.at[j]], add=True)
        return carry

    lax.fori_loop(0, NCHUNK, chunk, 0)
    plsc.subcore_barrier()

    pltpu.sync_copy(agg_sh.at[pl.ds(sid * NPT, NPT)],
                    outp.at[cid, pl.ds(sid * NPT, NPT)])


def _combine_body(degp_ref, p_ref, g_ref, b_ref, out_ref):
    s = degp_ref[0] + degp_ref[1]
    deg = jnp.sum(s, axis=1) * (1.0 / 16.0) + 1.0
    dinv = lax.rsqrt(deg)
    acc = (p_ref[0] + p_ref[1] + g_ref[...]) * dinv[:, None] + b_ref[0][None, :]
    out_ref[...] = jnp.broadcast_to(acc[None], out_ref.shape)


BN = 400


def kernel(x, edge_index, W, b):
    ei = edge_index.astype(jnp.int32)
    src = ei[0].reshape(NC, NS, NCHUNK, C)
    dst = ei[1].reshape(NC, NS, NCHUNK, C)

    degp = _deg_kernel(dst)

    g = pl.pallas_call(
        _scale_body,
        grid=(N // BN,),
        in_specs=[
            pl.BlockSpec((BN, D), lambda i: (i, 0)),
            pl.BlockSpec((D, D), lambda i: (0, 0)),
            pl.BlockSpec((NC, BN, 16), lambda i: (0, i, 0)),
        ],
        out_specs=pl.BlockSpec((BN, D), lambda i: (i, 0)),
        out_shape=jax.ShapeDtypeStruct((N, D), jnp.float32),
    )(x, W, degp)

    partials = _agg_kernel(src, dst, g)

    out = pl.pallas_call(
        _combine_body,
        grid=(N // BN,),
        in_specs=[
            pl.BlockSpec((NC, BN, 16), lambda i: (0, i, 0)),
            pl.BlockSpec((NC, BN, D), lambda i: (0, i, 0)),
            pl.BlockSpec((BN, D), lambda i: (i, 0)),
            pl.BlockSpec((1, D), lambda i: (0, 0)),
        ],
        out_specs=pl.BlockSpec((5, BN, D), lambda i: (0, i, 0)),
        out_shape=jax.ShapeDtypeStruct((5, N, D), jnp.float32),
    )(degp, partials, g, b.reshape(1, D))

    return out


# speedup vs baseline: 37.3103x; 1.4909x over previous
import functools

import jax
import jax.numpy as jnp
from jax import lax
from jax.experimental import pallas as pl
from jax.experimental.pallas import tpu as pltpu
from jax.experimental.pallas import tpu_sc as plsc

N = 10000
D = 128
E = 320000
NC = 2
NS = 16
NW = NC * NS
EPW = E // NW
C = 80
NCHUNK = EPW // C
NPT = N // NS
NBUF = 3

_mesh = plsc.VectorSubcoreMesh(core_axis_name="c", subcore_axis_name="s")
_sc_params = pltpu.CompilerParams(use_tc_tiling_on_sc=False)


@functools.partial(
    pl.kernel,
    out_type=jax.ShapeDtypeStruct((NC, N, 16), jnp.float32),
    mesh=_mesh,
    scratch_types=[
        pltpu.VMEM((NCHUNK, C), jnp.int32),
        pltpu.VMEM((C, 16), jnp.float32),
        pltpu.VMEM((NPT, 16), jnp.float32),
        pltpu.VMEM_SHARED((N, 16), jnp.float32),
    ],
    compiler_params=_sc_params,
)
def _deg_kernel(dst_hbm, outp, idxv, onesv, zerov, deg_sh):
    cid = lax.axis_index("c")
    sid = lax.axis_index("s")

    ones16 = jnp.ones((16,), jnp.float32)
    zeros16 = jnp.zeros((16,), jnp.float32)

    def fill_ones(i, carry):
        onesv[i] = ones16
        return carry

    lax.fori_loop(0, C, fill_ones, 0)

    def fill_zeros(i, carry):
        zerov[i] = zeros16
        return carry

    lax.fori_loop(0, NPT, fill_zeros, 0)

    pltpu.sync_copy(zerov, deg_sh.at[pl.ds(sid * NPT, NPT)])
    plsc.subcore_barrier()

    pltpu.sync_copy(dst_hbm.at[cid, sid], idxv)

    def chunk(j, carry):
        pltpu.sync_copy(onesv, deg_sh.at[idxv.at[j]], add=True)
        return carry

    lax.fori_loop(0, NCHUNK, chunk, 0)
    plsc.subcore_barrier()

    pltpu.sync_copy(deg_sh.at[pl.ds(sid * NPT, NPT)],
                    outp.at[cid, pl.ds(sid * NPT, NPT)])


def _scale_body(x_ref, w_ref, degp_ref, g_ref):
    s = degp_ref[0] + degp_ref[1]
    deg = jnp.sum(s, axis=1) * (1.0 / 16.0) + 1.0
    dinv = lax.rsqrt(deg)
    h = jnp.dot(x_ref[...], w_ref[...], preferred_element_type=jnp.float32)
    g_ref[...] = h * dinv[:, None]


@functools.partial(
    pl.kernel,
    out_type=jax.ShapeDtypeStruct((NC, N, D), jnp.float32),
    mesh=_mesh,
    scratch_types=[
        pltpu.VMEM((NCHUNK, C), jnp.int32),
        pltpu.VMEM((NCHUNK, C), jnp.int32),
        pltpu.VMEM((NBUF, C, D), jnp.float32),
        pltpu.VMEM_SHARED((N, D), jnp.float32),
        pltpu.SemaphoreType.DMA,
        pltpu.SemaphoreType.DMA,
    ],
    compiler_params=_sc_params,
)
def _agg_kernel(src_hbm, dst_hbm, g_hbm, zeros_hbm, outp, srcv, dstv, rows,
                agg_sh, sem_g, sem_s):
    cid = lax.axis_index("c")
    sid = lax.axis_index("s")

    pltpu.sync_copy(zeros_hbm, agg_sh.at[pl.ds(sid * NPT, NPT)])
    plsc.subcore_barrier()

    pltpu.sync_copy(src_hbm.at[cid, sid], srcv)
    pltpu.sync_copy(dst_hbm.at[cid, sid], dstv)

    for b in range(NBUF - 1):
        pltpu.async_copy(g_hbm.at[srcv.at[b]], rows.at[b], sem_g)

    def chunk(j, carry):
        buf = rows.at[lax.rem(j, NBUF)]
        pltpu.make_async_copy(g_hbm.at[srcv.at[j]], buf, sem_g).wait()
        pltpu.async_copy(buf, agg_sh.at[dstv.at[j]], sem_s, add=True)

        @pl.when(j >= 1)
        def _():
            pltpu.make_async_copy(rows.at[0], agg_sh.at[dstv.at[j]],
                                  sem_s).wait()

        @pl.when(j + NBUF - 1 < NCHUNK)
        def _():
            nxt = j + NBUF - 1
            pltpu.async_copy(g_hbm.at[srcv.at[nxt]],
                             rows.at[lax.rem(nxt, NBUF)], sem_g)

        return carry

    lax.fori_loop(0, NCHUNK, chunk, 0)
    pltpu.make_async_copy(rows.at[0], agg_sh.at[dstv.at[0]], sem_s).wait()
    plsc.subcore_barrier()

    pltpu.sync_copy(agg_sh.at[pl.ds(sid * NPT, NPT)],
                    outp.at[cid, pl.ds(sid * NPT, NPT)])


def _combine_body(degp_ref, p_ref, g_ref, b_ref, out_ref):
    s = degp_ref[0] + degp_ref[1]
    deg = jnp.sum(s, axis=1) * (1.0 / 16.0) + 1.0
    dinv = lax.rsqrt(deg)
    acc = (p_ref[0] + p_ref[1] + g_ref[...]) * dinv[:, None] + b_ref[0][None, :]
    out_ref[...] = jnp.broadcast_to(acc[None], out_ref.shape)


BN = 400


def kernel(x, edge_index, W, b):
    ei = edge_index.astype(jnp.int32)
    src = ei[0].reshape(NC, NS, NCHUNK, C)
    dst = ei[1].reshape(NC, NS, NCHUNK, C)

    degp = _deg_kernel(dst)

    g = pl.pallas_call(
        _scale_body,
        grid=(N // BN,),
        in_specs=[
            pl.BlockSpec((BN, D), lambda i: (i, 0)),
            pl.BlockSpec((D, D), lambda i: (0, 0)),
            pl.BlockSpec((NC, BN, 16), lambda i: (0, i, 0)),
        ],
        out_specs=pl.BlockSpec((BN, D), lambda i: (i, 0)),
        out_shape=jax.ShapeDtypeStruct((N, D), jnp.float32),
    )(x, W, degp)

    partials = _agg_kernel(src, dst, g, jnp.zeros((NPT, D), jnp.float32))

    out = pl.pallas_call(
        _combine_body,
        grid=(N // BN,),
        in_specs=[
            pl.BlockSpec((NC, BN, 16), lambda i: (0, i, 0)),
            pl.BlockSpec((NC, BN, D), lambda i: (0, i, 0)),
            pl.BlockSpec((BN, D), lambda i: (i, 0)),
            pl.BlockSpec((1, D), lambda i: (0, 0)),
        ],
        out_specs=pl.BlockSpec((5, BN, D), lambda i: (0, i, 0)),
        out_shape=jax.ShapeDtypeStruct((5, N, D), jnp.float32),
    )(degp, partials, g, b.reshape(1, D))

    return out


# speedup vs baseline: 43.0885x; 1.1549x over previous
import functools

import jax
import jax.numpy as jnp
from jax import lax
from jax.experimental import pallas as pl
from jax.experimental.pallas import tpu as pltpu
from jax.experimental.pallas import tpu_sc as plsc

N = 10000
D = 128
E = 320000
NC = 2
NS = 16
NW = NC * NS
EPW = E // NW
C = 80
NCHUNK = EPW // C
NPT = N // NS
NBUF = 3

_mesh = plsc.VectorSubcoreMesh(core_axis_name="c", subcore_axis_name="s")
_sc_params = pltpu.CompilerParams(use_tc_tiling_on_sc=False)


@functools.partial(
    pl.kernel,
    out_type=jax.ShapeDtypeStruct((NC, N, 16), jnp.float32),
    mesh=_mesh,
    scratch_types=[
        pltpu.VMEM((NCHUNK, C), jnp.int32),
        pltpu.VMEM((C, 16), jnp.float32),
        pltpu.VMEM((NPT, 16), jnp.float32),
        pltpu.VMEM_SHARED((N, 16), jnp.float32),
    ],
    compiler_params=_sc_params,
)
def _deg_kernel(ei_hbm, outp, idxv, onesv, zerov, deg_sh):
    cid = lax.axis_index("c")
    sid = lax.axis_index("s")

    ones16 = jnp.ones((16,), jnp.float32)
    zeros16 = jnp.zeros((16,), jnp.float32)

    def fill_ones(i, carry):
        onesv[i] = ones16
        return carry

    lax.fori_loop(0, C, fill_ones, 0)

    def fill_zeros(i, carry):
        zerov[i] = zeros16
        return carry

    lax.fori_loop(0, NPT, fill_zeros, 0)

    pltpu.sync_copy(zerov, deg_sh.at[pl.ds(sid * NPT, NPT)])
    plsc.subcore_barrier()

    pltpu.sync_copy(ei_hbm.at[1, cid, sid], idxv)

    def chunk(j, carry):
        pltpu.sync_copy(onesv, deg_sh.at[idxv.at[j]], add=True)
        return carry

    lax.fori_loop(0, NCHUNK, chunk, 0)
    plsc.subcore_barrier()

    pltpu.sync_copy(deg_sh.at[pl.ds(sid * NPT, NPT)],
                    outp.at[cid, pl.ds(sid * NPT, NPT)])


def _matmul_body(x_ref, w_ref, h_ref):
    h_ref[...] = jnp.dot(x_ref[...], w_ref[...],
                         preferred_element_type=jnp.float32)


def _scale_body(h_ref, degp_ref, g_ref):
    s = degp_ref[0] + degp_ref[1]
    deg = jnp.sum(s, axis=1) * (1.0 / 16.0) + 1.0
    dinv = lax.rsqrt(deg)
    g_ref[...] = h_ref[...] * dinv[:, None]


@functools.partial(
    pl.kernel,
    out_type=jax.ShapeDtypeStruct((NC, N, D), jnp.float32),
    mesh=_mesh,
    scratch_types=[
        pltpu.VMEM((NCHUNK, C), jnp.int32),
        pltpu.VMEM((NCHUNK, C), jnp.int32),
        pltpu.VMEM((NBUF, C, D), jnp.float32),
        pltpu.VMEM_SHARED((N, D), jnp.float32),
        pltpu.SemaphoreType.DMA,
        pltpu.SemaphoreType.DMA,
    ],
    compiler_params=_sc_params,
)
def _agg_kernel(ei_hbm, g_hbm, zeros_hbm, outp, srcv, dstv, rows,
                agg_sh, sem_g, sem_s):
    cid = lax.axis_index("c")
    sid = lax.axis_index("s")

    pltpu.sync_copy(zeros_hbm, agg_sh.at[pl.ds(sid * NPT, NPT)])
    plsc.subcore_barrier()

    pltpu.sync_copy(ei_hbm.at[0, cid, sid], srcv)
    pltpu.sync_copy(ei_hbm.at[1, cid, sid], dstv)

    for b in range(NBUF - 1):
        pltpu.async_copy(g_hbm.at[srcv.at[b]], rows.at[b], sem_g)

    def chunk(j, carry):
        buf = rows.at[lax.rem(j, NBUF)]
        pltpu.make_async_copy(g_hbm.at[srcv.at[j]], buf, sem_g).wait()
        pltpu.async_copy(buf, agg_sh.at[dstv.at[j]], sem_s, add=True)

        @pl.when(j >= 1)
        def _():
            pltpu.make_async_copy(rows.at[0], agg_sh.at[dstv.at[j]],
                                  sem_s).wait()

        @pl.when(j + NBUF - 1 < NCHUNK)
        def _():
            nxt = j + NBUF - 1
            pltpu.async_copy(g_hbm.at[srcv.at[nxt]],
                             rows.at[lax.rem(nxt, NBUF)], sem_g)

        return carry

    lax.fori_loop(0, NCHUNK, chunk, 0)
    pltpu.make_async_copy(rows.at[0], agg_sh.at[dstv.at[0]], sem_s).wait()
    plsc.subcore_barrier()

    pltpu.sync_copy(agg_sh.at[pl.ds(sid * NPT, NPT)],
                    outp.at[cid, pl.ds(sid * NPT, NPT)])


def _combine_body(degp_ref, p_ref, g_ref, b_ref, out_ref):
    s = degp_ref[0] + degp_ref[1]
    deg = jnp.sum(s, axis=1) * (1.0 / 16.0) + 1.0
    dinv = lax.rsqrt(deg)
    acc = (p_ref[0] + p_ref[1] + g_ref[...]) * dinv[:, None] + b_ref[0][None, :]
    out_ref[...] = jnp.broadcast_to(acc[None], out_ref.shape)


BN = 1000


def kernel(x, edge_index, W, b):
    ei = edge_index.astype(jnp.int32).reshape(2, NC, NS, NCHUNK, C)

    degp = _deg_kernel(ei)

    h = pl.pallas_call(
        _matmul_body,
        grid=(N // BN,),
        in_specs=[
            pl.BlockSpec((BN, D), lambda i: (i, 0)),
            pl.BlockSpec((D, D), lambda i: (0, 0)),
        ],
        out_specs=pl.BlockSpec((BN, D), lambda i: (i, 0)),
        out_shape=jax.ShapeDtypeStruct((N, D), jnp.float32),
    )(x, W)

    g = pl.pallas_call(
        _scale_body,
        grid=(N // BN,),
        in_specs=[
            pl.BlockSpec((BN, D), lambda i: (i, 0)),
            pl.BlockSpec((NC, BN, 16), lambda i: (0, i, 0)),
        ],
        out_specs=pl.BlockSpec((BN, D), lambda i: (i, 0)),
        out_shape=jax.ShapeDtypeStruct((N, D), jnp.float32),
    )(h, degp)

    partials = _agg_kernel(ei, g, jnp.zeros((NPT, D), jnp.float32))

    out = pl.pallas_call(
        _combine_body,
        grid=(N // BN,),
        in_specs=[
            pl.BlockSpec((NC, BN, 16), lambda i: (0, i, 0)),
            pl.BlockSpec((NC, BN, D), lambda i: (0, i, 0)),
            pl.BlockSpec((BN, D), lambda i: (i, 0)),
            pl.BlockSpec((1, D), lambda i: (0, 0)),
        ],
        out_specs=pl.BlockSpec((5, BN, D), lambda i: (0, i, 0)),
        out_shape=jax.ShapeDtypeStruct((5, N, D), jnp.float32),
    )(degp, partials, g, b.reshape(1, D))

    return out


# speedup vs baseline: 45.3425x; 1.0523x over previous
import functools

import jax
import jax.numpy as jnp
from jax import lax
from jax.experimental import pallas as pl
from jax.experimental.pallas import tpu as pltpu
from jax.experimental.pallas import tpu_sc as plsc

N = 10000
D = 128
E = 320000
NC = 2
NS = 16
NW = NC * NS
EPW = E // NW
C = 80
NCHUNK = EPW // C
NPT = N // NS
NBUF = 3
DEG_LAG = 8

_mesh = plsc.VectorSubcoreMesh(core_axis_name="c", subcore_axis_name="s")
_sc_params = pltpu.CompilerParams(use_tc_tiling_on_sc=False)


@functools.partial(
    pl.kernel,
    out_type=jax.ShapeDtypeStruct((NC, N, 16), jnp.float32),
    mesh=_mesh,
    scratch_types=[
        pltpu.VMEM((NCHUNK, C), jnp.int32),
        pltpu.VMEM((C, 16), jnp.float32),
        pltpu.VMEM((NPT, 16), jnp.float32),
        pltpu.VMEM_SHARED((N, 16), jnp.float32),
        pltpu.SemaphoreType.DMA,
    ],
    compiler_params=_sc_params,
)
def _deg_kernel(ei_hbm, outp, idxv, onesv, zerov, deg_sh, sem):
    cid = lax.axis_index("c")
    sid = lax.axis_index("s")

    ones16 = jnp.ones((16,), jnp.float32)
    zeros16 = jnp.zeros((16,), jnp.float32)

    def fill_ones(i, carry):
        onesv[i] = ones16
        return carry

    lax.fori_loop(0, C, fill_ones, 0)

    def fill_zeros(i, carry):
        zerov[i] = zeros16
        return carry

    lax.fori_loop(0, NPT, fill_zeros, 0)

    pltpu.sync_copy(zerov, deg_sh.at[pl.ds(sid * NPT, NPT)])
    plsc.subcore_barrier()

    pltpu.sync_copy(ei_hbm.at[1, cid, sid], idxv)

    def chunk(j, carry):
        pltpu.async_copy(onesv, deg_sh.at[idxv.at[j]], sem, add=True)

        @pl.when(j >= DEG_LAG)
        def _():
            pltpu.make_async_copy(onesv, deg_sh.at[idxv.at[0]], sem).wait()

        return carry

    lax.fori_loop(0, NCHUNK, chunk, 0)
    for _ in range(DEG_LAG):
        pltpu.make_async_copy(onesv, deg_sh.at[idxv.at[0]], sem).wait()
    plsc.subcore_barrier()

    pltpu.sync_copy(deg_sh.at[pl.ds(sid * NPT, NPT)],
                    outp.at[cid, pl.ds(sid * NPT, NPT)])


def _matmul_body(x_ref, w_ref, h_ref):
    h_ref[...] = jnp.dot(x_ref[...], w_ref[...],
                         preferred_element_type=jnp.float32)


def _scale_body(h_ref, degp_ref, g_ref):
    s = degp_ref[0] + degp_ref[1]
    deg = jnp.sum(s, axis=1) * (1.0 / 16.0) + 1.0
    dinv = lax.rsqrt(deg)
    g_ref[...] = h_ref[...] * dinv[:, None]


@functools.partial(
    pl.kernel,
    out_type=jax.ShapeDtypeStruct((NC, N, D), jnp.float32),
    mesh=_mesh,
    scratch_types=[
        pltpu.VMEM((NCHUNK, C), jnp.int32),
        pltpu.VMEM((NCHUNK, C), jnp.int32),
        pltpu.VMEM((NBUF, C, D), jnp.float32),
        pltpu.VMEM_SHARED((N, D), jnp.float32),
        pltpu.SemaphoreType.DMA,
        pltpu.SemaphoreType.DMA,
    ],
    compiler_params=_sc_params,
)
def _agg_kernel(ei_hbm, g_hbm, zeros_hbm, outp, srcv, dstv, rows,
                agg_sh, sem_g, sem_s):
    cid = lax.axis_index("c")
    sid = lax.axis_index("s")

    @pl.when(cid == 0)
    def _():
        pltpu.sync_copy(g_hbm.at[pl.ds(sid * NPT, NPT)],
                        agg_sh.at[pl.ds(sid * NPT, NPT)])

    @pl.when(cid == 1)
    def _():
        pltpu.sync_copy(zeros_hbm, agg_sh.at[pl.ds(sid * NPT, NPT)])

    plsc.subcore_barrier()

    pltpu.sync_copy(ei_hbm.at[0, cid, sid], srcv)
    pltpu.sync_copy(ei_hbm.at[1, cid, sid], dstv)

    for b in range(NBUF - 1):
        pltpu.async_copy(g_hbm.at[srcv.at[b]], rows.at[b], sem_g)

    def chunk(j, carry):
        buf = rows.at[lax.rem(j, NBUF)]
        pltpu.make_async_copy(g_hbm.at[srcv.at[j]], buf, sem_g).wait()
        pltpu.async_copy(buf, agg_sh.at[dstv.at[j]], sem_s, add=True)

        @pl.when(j >= 1)
        def _():
            pltpu.make_async_copy(rows.at[0], agg_sh.at[dstv.at[j]],
                                  sem_s).wait()

        @pl.when(j + NBUF - 1 < NCHUNK)
        def _():
            nxt = j + NBUF - 1
            pltpu.async_copy(g_hbm.at[srcv.at[nxt]],
                             rows.at[lax.rem(nxt, NBUF)], sem_g)

        return carry

    lax.fori_loop(0, NCHUNK, chunk, 0)
    pltpu.make_async_copy(rows.at[0], agg_sh.at[dstv.at[0]], sem_s).wait()
    plsc.subcore_barrier()

    pltpu.sync_copy(agg_sh.at[pl.ds(sid * NPT, NPT)],
                    outp.at[cid, pl.ds(sid * NPT, NPT)])


def _combine_body(degp_ref, p_ref, b_ref, out_ref):
    s = degp_ref[0] + degp_ref[1]
    deg = jnp.sum(s, axis=1) * (1.0 / 16.0) + 1.0
    dinv = lax.rsqrt(deg)
    acc = (p_ref[0] + p_ref[1]) * dinv[:, None] + b_ref[0][None, :]
    out_ref[...] = jnp.broadcast_to(acc[None], out_ref.shape)


BN = 1000


def kernel(x, edge_index, W, b):
    ei = edge_index.astype(jnp.int32).reshape(2, NC, NS, NCHUNK, C)

    degp = _deg_kernel(ei)

    h = pl.pallas_call(
        _matmul_body,
        grid=(N // BN,),
        in_specs=[
            pl.BlockSpec((BN, D), lambda i: (i, 0)),
            pl.BlockSpec((D, D), lambda i: (0, 0)),
        ],
        out_specs=pl.BlockSpec((BN, D), lambda i: (i, 0)),
        out_shape=jax.ShapeDtypeStruct((N, D), jnp.float32),
    )(x, W)

    g = pl.pallas_call(
        _scale_body,
        grid=(N // BN,),
        in_specs=[
            pl.BlockSpec((BN, D), lambda i: (i, 0)),
            pl.BlockSpec((NC, BN, 16), lambda i: (0, i, 0)),
        ],
        out_specs=pl.BlockSpec((BN, D), lambda i: (i, 0)),
        out_shape=jax.ShapeDtypeStruct((N, D), jnp.float32),
    )(h, degp)

    partials = _agg_kernel(ei, g, jnp.zeros((NPT, D), jnp.float32))

    out = pl.pallas_call(
        _combine_body,
        grid=(N // BN,),
        in_specs=[
            pl.BlockSpec((NC, BN, 16), lambda i: (0, i, 0)),
            pl.BlockSpec((NC, BN, D), lambda i: (0, i, 0)),
            pl.BlockSpec((1, D), lambda i: (0, 0)),
        ],
        out_specs=pl.BlockSpec((5, BN, D), lambda i: (0, i, 0)),
        out_shape=jax.ShapeDtypeStruct((5, N, D), jnp.float32),
    )(degp, partials, b.reshape(1, D))

    return out


# speedup vs baseline: 47.6795x; 1.0515x over previous
import functools

import jax
import jax.numpy as jnp
from jax import lax
from jax.experimental import pallas as pl
from jax.experimental.pallas import tpu as pltpu
from jax.experimental.pallas import tpu_sc as plsc

N = 10000
D = 128
E = 320000
NC = 2
NS = 16
NW = NC * NS
EPW = E // NW
C = 80
NCHUNK = EPW // C
NPT = N // NS
DW = 8
NBUF = 3
DEG_LAG = 12

_mesh = plsc.VectorSubcoreMesh(core_axis_name="c", subcore_axis_name="s")
_sc_params = pltpu.CompilerParams(use_tc_tiling_on_sc=False)


@functools.partial(
    pl.kernel,
    out_type=jax.ShapeDtypeStruct((NC, N, DW), jnp.float32),
    mesh=_mesh,
    scratch_types=[
        pltpu.VMEM((NCHUNK, C), jnp.int32),
        pltpu.VMEM((C, DW), jnp.float32),
        pltpu.VMEM((NPT + 1, DW), jnp.float32),
        pltpu.VMEM_SHARED((N, DW), jnp.float32),
        pltpu.SemaphoreType.DMA,
    ],
    compiler_params=_sc_params,
)
def _deg_kernel(ei_hbm, outp, idxv, onesv, zerov, deg_sh, sem):
    cid = lax.axis_index("c")
    sid = lax.axis_index("s")

    ones2x8 = jnp.ones((2, 8), jnp.float32)
    zeros2x8 = jnp.zeros((2, 8), jnp.float32)

    def fill_ones(i, carry):
        onesv[pl.ds(2 * i, 2)] = ones2x8
        return carry

    lax.fori_loop(0, C // 2, fill_ones, 0)

    def fill_zeros(i, carry):
        zerov[pl.ds(2 * i, 2)] = zeros2x8
        return carry

    lax.fori_loop(0, (NPT + 1) // 2, fill_zeros, 0)

    pltpu.sync_copy(zerov.at[pl.ds(0, NPT)], deg_sh.at[pl.ds(sid * NPT, NPT)])
    plsc.subcore_barrier()

    pltpu.sync_copy(ei_hbm.at[1, cid, sid], idxv)

    def chunk(j, carry):
        pltpu.async_copy(onesv, deg_sh.at[idxv.at[j]], sem, add=True)

        @pl.when(j >= DEG_LAG)
        def _():
            pltpu.make_async_copy(onesv, deg_sh.at[idxv.at[0]], sem).wait()

        return carry

    lax.fori_loop(0, NCHUNK, chunk, 0)
    for _ in range(DEG_LAG):
        pltpu.make_async_copy(onesv, deg_sh.at[idxv.at[0]], sem).wait()
    plsc.subcore_barrier()

    pltpu.sync_copy(deg_sh.at[pl.ds(sid * NPT, NPT)],
                    outp.at[cid, pl.ds(sid * NPT, NPT)])


def _matmul_body(x_ref, w_ref, h_ref):
    h_ref[...] = jnp.dot(x_ref[...], w_ref[...],
                         preferred_element_type=jnp.float32)


def _scale_body(h_ref, degp_ref, g_ref):
    s = degp_ref[0] + degp_ref[1]
    deg = jnp.sum(s, axis=1) * (1.0 / DW) + 1.0
    dinv = lax.rsqrt(deg)
    g_ref[...] = h_ref[...] * dinv[:, None]


@functools.partial(
    pl.kernel,
    out_type=jax.ShapeDtypeStruct((NC, N, D), jnp.float32),
    mesh=_mesh,
    scratch_types=[
        pltpu.VMEM((NCHUNK, C), jnp.int32),
        pltpu.VMEM((NCHUNK, C), jnp.int32),
        pltpu.VMEM((NBUF, C, D), jnp.float32),
        pltpu.VMEM_SHARED((N, D), jnp.float32),
        pltpu.SemaphoreType.DMA,
        pltpu.SemaphoreType.DMA,
    ],
    compiler_params=_sc_params,
)
def _agg_kernel(ei_hbm, g_hbm, zeros_hbm, outp, srcv, dstv, rows,
                agg_sh, sem_g, sem_s):
    cid = lax.axis_index("c")
    sid = lax.axis_index("s")

    @pl.when(cid == 0)
    def _():
        pltpu.sync_copy(g_hbm.at[pl.ds(sid * NPT, NPT)],
                        agg_sh.at[pl.ds(sid * NPT, NPT)])

    @pl.when(cid == 1)
    def _():
        pltpu.sync_copy(zeros_hbm, agg_sh.at[pl.ds(sid * NPT, NPT)])

    plsc.subcore_barrier()

    pltpu.sync_copy(ei_hbm.at[0, cid, sid], srcv)
    pltpu.sync_copy(ei_hbm.at[1, cid, sid], dstv)

    for b in range(NBUF - 1):
        pltpu.async_copy(g_hbm.at[srcv.at[b]], rows.at[b], sem_g)

    def chunk(j, carry):
        buf = rows.at[lax.rem(j, NBUF)]
        pltpu.make_async_copy(g_hbm.at[srcv.at[j]], buf, sem_g).wait()
        pltpu.async_copy(buf, agg_sh.at[dstv.at[j]], sem_s, add=True)

        @pl.when(j >= 1)
        def _():
            pltpu.make_async_copy(rows.at[0], agg_sh.at[dstv.at[j]],
                                  sem_s).wait()

        @pl.when(j + NBUF - 1 < NCHUNK)
        def _():
            nxt = j + NBUF - 1
            pltpu.async_copy(g_hbm.at[srcv.at[nxt]],
                             rows.at[lax.rem(nxt, NBUF)], sem_g)

        return carry

    lax.fori_loop(0, NCHUNK, chunk, 0)
    pltpu.make_async_copy(rows.at[0], agg_sh.at[dstv.at[0]], sem_s).wait()
    plsc.subcore_barrier()

    pltpu.sync_copy(agg_sh.at[pl.ds(sid * NPT, NPT)],
                    outp.at[cid, pl.ds(sid * NPT, NPT)])


def _combine_body(degp_ref, p_ref, b_ref, out_ref):
    s = degp_ref[0] + degp_ref[1]
    deg = jnp.sum(s, axis=1) * (1.0 / DW) + 1.0
    dinv = lax.rsqrt(deg)
    acc = (p_ref[0] + p_ref[1]) * dinv[:, None] + b_ref[0][None, :]
    out_ref[...] = jnp.broadcast_to(acc[None], out_ref.shape)


BN = 2000


def kernel(x, edge_index, W, b):
    ei = edge_index.astype(jnp.int32).reshape(2, NC, NS, NCHUNK, C)

    degp = _deg_kernel(ei)

    h = pl.pallas_call(
        _matmul_body,
        grid=(N // BN,),
        in_specs=[
            pl.BlockSpec((BN, D), lambda i: (i, 0)),
            pl.BlockSpec((D, D), lambda i: (0, 0)),
        ],
        out_specs=pl.BlockSpec((BN, D), lambda i: (i, 0)),
        out_shape=jax.ShapeDtypeStruct((N, D), jnp.float32),
    )(x, W)

    g = pl.pallas_call(
        _scale_body,
        grid=(N // BN,),
        in_specs=[
            pl.BlockSpec((BN, D), lambda i: (i, 0)),
            pl.BlockSpec((NC, BN, DW), lambda i: (0, i, 0)),
        ],
        out_specs=pl.BlockSpec((BN, D), lambda i: (i, 0)),
        out_shape=jax.ShapeDtypeStruct((N, D), jnp.float32),
    )(h, degp)

    partials = _agg_kernel(ei, g, jnp.zeros((NPT, D), jnp.float32))

    out = pl.pallas_call(
        _combine_body,
        grid=(N // BN,),
        in_specs=[
            pl.BlockSpec((NC, BN, DW), lambda i: (0, i, 0)),
            pl.BlockSpec((NC, BN, D), lambda i: (0, i, 0)),
            pl.BlockSpec((1, D), lambda i: (0, 0)),
        ],
        out_specs=pl.BlockSpec((5, BN, D), lambda i: (0, i, 0)),
        out_shape=jax.ShapeDtypeStruct((5, N, D), jnp.float32),
    )(degp, partials, b.reshape(1, D))

    return out


# speedup vs baseline: 47.8342x; 1.0032x over previous
import functools

import jax
import jax.numpy as jnp
from jax import lax
from jax.experimental import pallas as pl
from jax.experimental.pallas import tpu as pltpu
from jax.experimental.pallas import tpu_sc as plsc

N = 10000
D = 128
E = 320000
NC = 2
NS = 16
NW = NC * NS
EPW = E // NW
C = 80
NCHUNK = EPW // C
NPT = N // NS
DW = 4
NBUF = 3
DEG_LAG = 12

_mesh = plsc.VectorSubcoreMesh(core_axis_name="c", subcore_axis_name="s")
_sc_params = pltpu.CompilerParams(use_tc_tiling_on_sc=False)


@functools.partial(
    pl.kernel,
    out_type=jax.ShapeDtypeStruct((NC, N, DW), jnp.float32),
    mesh=_mesh,
    scratch_types=[
        pltpu.VMEM((NCHUNK, C), jnp.int32),
        pltpu.VMEM((C, DW), jnp.float32),
        pltpu.VMEM((NPT + 3, DW), jnp.float32),
        pltpu.VMEM_SHARED((N, DW), jnp.float32),
        pltpu.SemaphoreType.DMA,
    ],
    compiler_params=_sc_params,
)
def _deg_kernel(ei_hbm, outp, idxv, onesv, zerov, deg_sh, sem):
    cid = lax.axis_index("c")
    sid = lax.axis_index("s")

    ones4x4 = jnp.ones((4, 4), jnp.float32)
    zeros4x4 = jnp.zeros((4, 4), jnp.float32)

    def fill_ones(i, carry):
        onesv[pl.ds(4 * i, 4)] = ones4x4
        return carry

    lax.fori_loop(0, C // 4, fill_ones, 0)

    def fill_zeros(i, carry):
        zerov[pl.ds(4 * i, 4)] = zeros4x4
        return carry

    lax.fori_loop(0, (NPT + 3) // 4, fill_zeros, 0)

    pltpu.sync_copy(zerov.at[pl.ds(0, NPT)], deg_sh.at[pl.ds(sid * NPT, NPT)])
    plsc.subcore_barrier()

    pltpu.sync_copy(ei_hbm.at[1, cid, sid], idxv)

    def chunk(j, carry):
        pltpu.async_copy(onesv, deg_sh.at[idxv.at[j]], sem, add=True)

        @pl.when(j >= DEG_LAG)
        def _():
            pltpu.make_async_copy(onesv, deg_sh.at[idxv.at[0]], sem).wait()

        return carry

    lax.fori_loop(0, NCHUNK, chunk, 0)
    for _ in range(DEG_LAG):
        pltpu.make_async_copy(onesv, deg_sh.at[idxv.at[0]], sem).wait()
    plsc.subcore_barrier()

    pltpu.sync_copy(deg_sh.at[pl.ds(sid * NPT, NPT)],
                    outp.at[cid, pl.ds(sid * NPT, NPT)])


def _matmul_body(x_ref, w_ref, h_ref):
    h_ref[...] = jnp.dot(x_ref[...], w_ref[...],
                         preferred_element_type=jnp.float32)


def _scale_body(h_ref, degp_ref, g_ref):
    s = degp_ref[0] + degp_ref[1]
    deg = jnp.sum(s, axis=1) * (1.0 / DW) + 1.0
    dinv = lax.rsqrt(deg)
    g_ref[...] = h_ref[...] * dinv[:, None]


@functools.partial(
    pl.kernel,
    out_type=jax.ShapeDtypeStruct((NC, N, D), jnp.float32),
    mesh=_mesh,
    scratch_types=[
        pltpu.VMEM((NCHUNK, C), jnp.int32),
        pltpu.VMEM((NCHUNK, C), jnp.int32),
        pltpu.VMEM((NBUF, C, D), jnp.float32),
        pltpu.VMEM_SHARED((N, D), jnp.float32),
        pltpu.SemaphoreType.DMA,
        pltpu.SemaphoreType.DMA,
    ],
    compiler_params=_sc_params,
)
def _agg_kernel(ei_hbm, g_hbm, zeros_hbm, outp, srcv, dstv, rows,
                agg_sh, sem_g, sem_s):
    cid = lax.axis_index("c")
    sid = lax.axis_index("s")

    @pl.when(cid == 0)
    def _():
        pltpu.sync_copy(g_hbm.at[pl.ds(sid * NPT, NPT)],
                        agg_sh.at[pl.ds(sid * NPT, NPT)])

    @pl.when(cid == 1)
    def _():
        pltpu.sync_copy(zeros_hbm, agg_sh.at[pl.ds(sid * NPT, NPT)])

    plsc.subcore_barrier()

    pltpu.sync_copy(ei_hbm.at[0, cid, sid], srcv)
    pltpu.sync_copy(ei_hbm.at[1, cid, sid], dstv)

    for b in range(NBUF - 1):
        pltpu.async_copy(g_hbm.at[srcv.at[b]], rows.at[b], sem_g)

    def chunk(j, carry):
        buf = rows.at[lax.rem(j, NBUF)]
        pltpu.make_async_copy(g_hbm.at[srcv.at[j]], buf, sem_g).wait()
        pltpu.async_copy(buf, agg_sh.at[dstv.at[j]], sem_s, add=True)

        @pl.when(j >= 1)
        def _():
            pltpu.make_async_copy(rows.at[0], agg_sh.at[dstv.at[j]],
                                  sem_s).wait()

        @pl.when(j + NBUF - 1 < NCHUNK)
        def _():
            nxt = j + NBUF - 1
            pltpu.async_copy(g_hbm.at[srcv.at[nxt]],
                             rows.at[lax.rem(nxt, NBUF)], sem_g)

        return carry

    lax.fori_loop(0, NCHUNK, chunk, 0)
    pltpu.make_async_copy(rows.at[0], agg_sh.at[dstv.at[0]], sem_s).wait()
    plsc.subcore_barrier()

    pltpu.sync_copy(agg_sh.at[pl.ds(sid * NPT, NPT)],
                    outp.at[cid, pl.ds(sid * NPT, NPT)])


def _combine_body(degp_ref, p_ref, b_ref, out_ref):
    s = degp_ref[0] + degp_ref[1]
    deg = jnp.sum(s, axis=1) * (1.0 / DW) + 1.0
    dinv = lax.rsqrt(deg)
    acc = (p_ref[0] + p_ref[1]) * dinv[:, None] + b_ref[0][None, :]
    out_ref[...] = jnp.broadcast_to(acc[None], out_ref.shape)


BN = 2000


def kernel(x, edge_index, W, b):
    ei = edge_index.astype(jnp.int32).reshape(2, NC, NS, NCHUNK, C)

    degp = _deg_kernel(ei)

    h = pl.pallas_call(
        _matmul_body,
        grid=(N // BN,),
        in_specs=[
            pl.BlockSpec((BN, D), lambda i: (i, 0)),
            pl.BlockSpec((D, D), lambda i: (0, 0)),
        ],
        out_specs=pl.BlockSpec((BN, D), lambda i: (i, 0)),
        out_shape=jax.ShapeDtypeStruct((N, D), jnp.float32),
    )(x, W)

    g = pl.pallas_call(
        _scale_body,
        grid=(N // BN,),
        in_specs=[
            pl.BlockSpec((BN, D), lambda i: (i, 0)),
            pl.BlockSpec((NC, BN, DW), lambda i: (0, i, 0)),
        ],
        out_specs=pl.BlockSpec((BN, D), lambda i: (i, 0)),
        out_shape=jax.ShapeDtypeStruct((N, D), jnp.float32),
    )(h, degp)

    partials = _agg_kernel(ei, g, jnp.zeros((NPT, D), jnp.float32))

    out = pl.pallas_call(
        _combine_body,
        grid=(N // BN,),
        in_specs=[
            pl.BlockSpec((NC, BN, DW), lambda i: (0, i, 0)),
            pl.BlockSpec((NC, BN, D), lambda i: (0, i, 0)),
            pl.BlockSpec((1, D), lambda i: (0, 0)),
        ],
        out_specs=pl.BlockSpec((5, BN, D), lambda i: (0, i, 0)),
        out_shape=jax.ShapeDtypeStruct((5, N, D), jnp.float32),
    )(degp, partials, b.reshape(1, D))

    return out
